# baseline (speedup 1.0000x reference)
import functools

import jax
import jax.numpy as jnp
from jax import lax
from jax.experimental import pallas as pl
from jax.experimental.pallas import tpu as pltpu
from jax.experimental.pallas import tpu_sc as plsc

_B = 16384
_D = 32
_NC = 2
_NS = 16
_NW = _NC * _NS
_BPW = _B // _NW
_CH = 128
_NCHUNK = _BPW // _CH


def _sc_body(uidx_hbm, iidx_hbm, ut4_hbm, it4_hbm, out_hbm,
             uidx_v, iidx_v, qu_v, qi_v, u4_v, i4_v, prod_v, s_u, s_i):
    c = lax.axis_index("c")
    s = lax.axis_index("s")
    wid = s * _NC + c
    base = wid * _BPW

    pltpu.sync_copy(uidx_hbm.at[pl.ds(base, _BPW)], uidx_v.at[pl.ds(0, _BPW)])
    pltpu.sync_copy(iidx_hbm.at[pl.ds(base, _BPW)], iidx_v.at[pl.ds(0, _BPW)])

    def qchunk(t, carry):
        qu_v[pl.ds(t * 16, 16)] = lax.shift_right_logical(
            uidx_v[pl.ds(t * 16, 16)], 2)
        qi_v[pl.ds(t * 16, 16)] = lax.shift_right_logical(
            iidx_v[pl.ds(t * 16, 16)], 2)
        return carry

    lax.fori_loop(0, _BPW // 16, qchunk, 0)

    for h in range(_NCHUNK):
        cu = pltpu.async_copy(ut4_hbm.at[qu_v.at[pl.ds(h * _CH, _CH)]],
                              u4_v, s_u)
        ci = pltpu.async_copy(it4_hbm.at[qi_v.at[pl.ds(h * _CH, _CH)]],
                              i4_v, s_i)
        cu.wait()
        ci.wait()

        def row(r, carry):
            au = uidx_v[pl.ds(h * _CH + r, 16)][0]
            ai = iidx_v[pl.ds(h * _CH + r, 16)][0]
            ou = (au & 3) * _D
            oi = (ai & 3) * _D
            pbase = (h * _CH + r) * _D
            prod_v[pl.ds(pbase, 16)] = (
                u4_v[r, pl.ds(ou, 16)] * i4_v[r, pl.ds(oi, 16)])
            prod_v[pl.ds(pbase + 16, 16)] = (
                u4_v[r, pl.ds(ou + 16, 16)] * i4_v[r, pl.ds(oi + 16, 16)])
            return carry

        lax.fori_loop(0, _CH, row, 0)

    pltpu.sync_copy(prod_v, out_hbm.at[pl.ds(wid * (_BPW * _D), _BPW * _D)])


_sc_gather_prod = pl.kernel(
    _sc_body,
    out_type=jax.ShapeDtypeStruct((_B * _D,), jnp.float32),
    mesh=plsc.VectorSubcoreMesh(core_axis_name="c", subcore_axis_name="s"),
    scratch_types=[
        pltpu.VMEM((_BPW + 16,), jnp.int32),
        pltpu.VMEM((_BPW + 16,), jnp.int32),
        pltpu.VMEM((_BPW,), jnp.int32),
        pltpu.VMEM((_BPW,), jnp.int32),
        pltpu.VMEM((_CH, 4 * _D), jnp.float32),
        pltpu.VMEM((_CH, 4 * _D), jnp.float32),
        pltpu.VMEM((_BPW * _D,), jnp.float32),
        pltpu.SemaphoreType.DMA,
        pltpu.SemaphoreType.DMA,
    ],
    name="sc_gather_prod",
)


def _tc_body(p_ref, wt4_ref, bt4_ref, wo4_ref, bo_ref, out_ref):
    h = jnp.dot(p_ref[...], wt4_ref[...], preferred_element_type=jnp.float32)
    h = jnp.maximum(h + bt4_ref[...], 0.0)
    out_ref[...] = jnp.dot(h, wo4_ref[...],
                           preferred_element_type=jnp.float32) + bo_ref[0, 0]


_tc_mlp = pl.pallas_call(
    _tc_body,
    out_shape=jax.ShapeDtypeStruct((_B // 4, 4), jnp.float32),
    name="tc_mlp",
)


def kernel(user_idx, item_idx, user_table, item_table, W_t, b_t, W_o, b_o):
    ut4 = user_table.reshape(-1, 4 * _D)
    it4 = item_table.reshape(-1, 4 * _D)
    prod_flat = _sc_gather_prod(user_idx.astype(jnp.int32),
                                item_idx.astype(jnp.int32),
                                ut4, it4)
    prod4 = prod_flat.reshape(_B // 4, 4 * _D)
    eye4 = jnp.eye(4, dtype=jnp.float32)
    wt4 = jnp.kron(eye4, W_t)
    wo4 = jnp.kron(eye4, W_o)
    bt4 = jnp.tile(b_t, 4).reshape(1, 4 * _D)
    pred = _tc_mlp(prod4, wt4, bt4, wo4, b_o.reshape(1, 1))
    return pred.reshape(_B)


# speedup vs baseline: 3.9210x; 3.9210x over previous
import functools

import jax
import jax.numpy as jnp
from jax import lax
from jax.experimental import pallas as pl
from jax.experimental.pallas import tpu as pltpu
from jax.experimental.pallas import tpu_sc as plsc

_B = 16384
_D = 32
_NC = 2
_NS = 16
_NW = _NC * _NS
_BPW = _B // _NW
_NBUF = 8


def _scal(ref, pos):
    return ref[pl.ds(pos, 16)][0]


def _sc_body(uidx_hbm, iidx_hbm, utT_hbm, itT_hbm, out_hbm,
             uidx_v, iidx_v, prod_v, bufs_and_sems):
    c = lax.axis_index("c")
    s = lax.axis_index("s")
    wid = s * _NC + c
    base = wid * _BPW

    ubufs = bufs_and_sems[:_NBUF]
    ibufs = bufs_and_sems[_NBUF:2 * _NBUF]
    usems = bufs_and_sems[2 * _NBUF:3 * _NBUF]
    isems = bufs_and_sems[3 * _NBUF:]

    pltpu.sync_copy(uidx_hbm.at[pl.ds(base, _BPW)], uidx_v.at[pl.ds(0, _BPW)])
    pltpu.sync_copy(iidx_hbm.at[pl.ds(base, _BPW)], iidx_v.at[pl.ds(0, _BPW)])

    def issue(n, b):
        tu = (lax.shift_right_logical(_scal(uidx_v, n), 7)) * 128
        ti = (lax.shift_right_logical(_scal(iidx_v, n), 7)) * 128
        pltpu.async_copy(utT_hbm.at[:, pl.ds(pl.multiple_of(tu, 128), 128)],
                         ubufs[b], usems[b])
        pltpu.async_copy(itT_hbm.at[:, pl.ds(pl.multiple_of(ti, 128), 128)],
                         ibufs[b], isems[b])

    rows_lo = lax.iota(jnp.int32, 16)
    rows_hi = rows_lo + 16

    def consume(n, b):
        pltpu.make_async_copy(utT_hbm.at[:, pl.ds(0, 128)], ubufs[b],
                              usems[b]).wait()
        pltpu.make_async_copy(itT_hbm.at[:, pl.ds(0, 128)], ibufs[b],
                              isems[b]).wait()
        lu = _scal(uidx_v, n) & 127
        li = _scal(iidx_v, n) & 127
        cu = jnp.full((16,), lu, dtype=jnp.int32)
        ci = jnp.full((16,), li, dtype=jnp.int32)
        u0 = plsc.load_gather(ubufs[b], [rows_lo, cu])
        u1 = plsc.load_gather(ubufs[b], [rows_hi, cu])
        i0 = plsc.load_gather(ibufs[b], [rows_lo, ci])
        i1 = plsc.load_gather(ibufs[b], [rows_hi, ci])
        prod_v[pl.ds(n * _D, 16)] = u0 * i0
        prod_v[pl.ds(n * _D + 16, 16)] = u1 * i1

    for b in range(_NBUF):
        issue(b, b)

    def ring(i, carry):
        for b in range(_NBUF):
            n = i * _NBUF + b
            consume(n, b)

            @pl.when(n < _BPW - _NBUF)
            def _():
                issue(n + _NBUF, b)
        return carry

    lax.fori_loop(0, _BPW // _NBUF, ring, 0)

    pltpu.sync_copy(prod_v, out_hbm.at[pl.ds(wid * (_BPW * _D), _BPW * _D)])


def _sc_entry(uidx_hbm, iidx_hbm, utT_hbm, itT_hbm, out_hbm,
              uidx_v, iidx_v, prod_v, *bufs_and_sems):
    _sc_body(uidx_hbm, iidx_hbm, utT_hbm, itT_hbm, out_hbm,
             uidx_v, iidx_v, prod_v, bufs_and_sems)


_sc_gather_prod = pl.kernel(
    _sc_entry,
    out_type=jax.ShapeDtypeStruct((_B * _D,), jnp.float32),
    mesh=plsc.VectorSubcoreMesh(core_axis_name="c", subcore_axis_name="s"),
    scratch_types=(
        [pltpu.VMEM((_BPW + 16,), jnp.int32),
         pltpu.VMEM((_BPW + 16,), jnp.int32),
         pltpu.VMEM((_BPW * _D,), jnp.float32)]
        + [pltpu.VMEM((_D, 128), jnp.float32)] * (2 * _NBUF)
        + [pltpu.SemaphoreType.DMA] * (2 * _NBUF)
    ),
    compiler_params=pltpu.CompilerParams(needs_layout_passes=False),
    name="sc_gather_prod",
)


def _tc_body(p_ref, wt4_ref, bt4_ref, wo4_ref, bo_ref, out_ref):
    h = jnp.dot(p_ref[...], wt4_ref[...], preferred_element_type=jnp.float32)
    h = jnp.maximum(h + bt4_ref[...], 0.0)
    out_ref[...] = jnp.dot(h, wo4_ref[...],
                           preferred_element_type=jnp.float32) + bo_ref[0, 0]


_tc_mlp = pl.pallas_call(
    _tc_body,
    out_shape=jax.ShapeDtypeStruct((_B // 4, 4), jnp.float32),
    name="tc_mlp",
)


def kernel(user_idx, item_idx, user_table, item_table, W_t, b_t, W_o, b_o):
    prod_flat = _sc_gather_prod(user_idx.astype(jnp.int32),
                                item_idx.astype(jnp.int32),
                                user_table.T, item_table.T)
    prod4 = prod_flat.reshape(_B // 4, 4 * _D)
    eye4 = jnp.eye(4, dtype=jnp.float32)
    wt4 = jnp.kron(eye4, W_t)
    wo4 = jnp.kron(eye4, W_o)
    bt4 = jnp.tile(b_t, 4).reshape(1, 4 * _D)
    pred = _tc_mlp(prod4, wt4, bt4, wo4, b_o.reshape(1, 1))
    return pred.reshape(_B)
